# Initial kernel scaffold; baseline (speedup 1.0000x reference)
import jax
import jax.numpy as jnp
from jax.experimental import pallas as pl


def kernel(users, pos_items, pos_item_offsets, items, user_table, item_table):
    raise NotImplementedError("write your pallas kernel here")


# speedup vs baseline: 48.2261x; 48.2261x over previous
import jax
import jax.numpy as jnp
from jax import lax
from jax.experimental import pallas as pl
from jax.experimental.pallas import tpu as pltpu
from jax.experimental.pallas import tpu_sc as plsc

NC = 2
NS = 16
L = 16
NW = NC * NS
B = 16384
NPOS = 819200
D = 32
UPW = B // NW
T = 512
SUB = 128
NSUB = T // SUB
NU = UPW // SUB
NG = T // L
UG = UPW // L
G = 0.5


def _rsqrt16(x):
    i = plsc.bitcast(x, jnp.int32)
    y = plsc.bitcast(jnp.int32(0x5F3759DF) - (i >> 1), jnp.float32)
    h = x * 0.5
    for _ in range(3):
        y = y * (1.5 - h * y * y)
    return y


def _body(users_h, pos_h, offs_h, items_h, ut_h, it_h, out_h,
          offs_v, idx_v, buf_v, sums_v, marks_v, uidx_v, iidx_v,
          uemb_v, iemb_v, scores_v, gsem, esem):
    cid = lax.axis_index("c")
    sid = lax.axis_index("s")
    wid = sid * NC + cid
    u0 = pl.multiple_of(wid * UPW, UPW)
    lanes = lax.iota(jnp.int32, L)

    pltpu.sync_copy(offs_h.at[pl.ds(u0, UPW)], offs_v.at[pl.ds(0, UPW)])
    nxt_start = pl.multiple_of(jnp.minimum(u0 + UPW, B - 8), 8)
    pltpu.sync_copy(offs_h.at[pl.ds(nxt_start, 8)], offs_v.at[pl.ds(UPW, 8)])
    nxt0 = offs_v[pl.ds(UPW, L)][0]
    e_w = jnp.where(wid == NW - 1, NPOS, nxt0)
    offs_v[pl.ds(UPW, L)] = jnp.full((L,), e_w, jnp.int32)
    s_w = offs_v[pl.ds(0, L)][0]
    s128 = pl.multiple_of(s_w & (-128), 128)

    for k in range(NU):
        pltpu.sync_copy(users_h.at[pl.ds(pl.multiple_of(u0 + k * SUB, 8), SUB)],
                        uidx_v.at[k])
        pltpu.sync_copy(items_h.at[pl.ds(pl.multiple_of(u0 + k * SUB, 8), SUB)],
                        iidx_v.at[k])
    ecopies = []
    for k in range(NU):
        ecopies.append(pltpu.async_copy(
            ut_h.at[uidx_v.at[k]], uemb_v.at[pl.ds(k * SUB, SUB)], esem))
        ecopies.append(pltpu.async_copy(
            it_h.at[iidx_v.at[k]], iemb_v.at[pl.ds(k * SUB, SUB)], esem))

    accz = jnp.zeros((L,), jnp.float32)
    zi = jnp.zeros((L,), jnp.int32)
    ones = jnp.full((L,), 1, jnp.int32)

    def _zero(u, _):
        sums_v[u, pl.ds(0, L)] = accz
        sums_v[u, pl.ds(L, L)] = accz
        return 0

    lax.fori_loop(0, UPW, _zero, 0)

    nchunks = (e_w - s128 + (T - 1)) // T

    def _chunk(c, carry):
        base = s128 + c * T
        for k in range(NSUB):
            off_k = pl.multiple_of(
                jnp.minimum(base + k * SUB, NPOS - SUB), SUB)
            pltpu.sync_copy(pos_h.at[pl.ds(off_k, SUB)], idx_v.at[k])
        gc = [pltpu.async_copy(it_h.at[idx_v.at[k]],
                               buf_v.at[pl.ds(k * SUB, SUB)], gsem)
              for k in range(NSUB)]
        hi_c = jnp.minimum(e_w, base + T)

        for g in range(NG):
            marks_v[pl.ds(g * L, L)] = zi
        for ug in range(UG):
            o_u = offs_v[pl.ds(ug * L, L)]
            m = (o_u >= base) & (o_u < base + T)
            pos = jnp.clip(o_u - base, 0, T - 1)
            plsc.addupdate_scatter(marks_v, [pos], ones, mask=m)

        for g_ in gc:
            g_.wait()

        for g in range(NG):
            rv = base + g * L + lanes
            cs = plsc.cumsum(marks_v[pl.ds(g * L, L)])
            seg = carry + cs - 1
            carry = carry + cs[L - 1]
            m = (rv >= s_w) & (rv < hi_c)
            segc = jnp.clip(seg, 0, UPW - 1)
            rows16 = jnp.full((L,), g * L, jnp.int32) + lanes
            for d in range(D):
                dd = jnp.full((L,), d, jnp.int32)
                vals = plsc.load_gather(buf_v, [rows16, dd])
                plsc.addupdate_scatter(sums_v, [segc, dd], vals, mask=m)
        return carry

    lax.fori_loop(0, nchunks, _chunk, jnp.int32(0))

    for e in ecopies:
        e.wait()

    def _group(i, _):
        rows = i * L + lanes
        cnt = offs_v[pl.ds(i * L + 1, L)] - offs_v[pl.ds(i * L, L)]
        rec = 1.0 / jnp.maximum(cnt.astype(jnp.float32), 1.0)
        dotv = accz
        na2 = accz
        nb2 = accz
        for d in range(D):
            dd = jnp.full((L,), d, jnp.int32)
            mu = plsc.load_gather(sums_v, [rows, dd]) * rec
            uu = plsc.load_gather(uemb_v, [rows, dd])
            ii = plsc.load_gather(iemb_v, [rows, dd])
            a = G * uu + (1.0 - G) * mu
            dotv = dotv + a * ii
            na2 = na2 + a * a
            nb2 = nb2 + ii * ii
        prod = jnp.maximum(na2, 1e-16) * jnp.maximum(nb2, 1e-16)
        scores_v[pl.ds(i * L, L)] = dotv * _rsqrt16(prod)
        return 0

    lax.fori_loop(0, UG, _group, 0)
    pltpu.sync_copy(scores_v, out_h.at[pl.ds(u0, UPW)])


@jax.jit
def kernel(users, pos_items, pos_item_offsets, items, user_table, item_table):
    mesh = plsc.VectorSubcoreMesh(core_axis_name="c", subcore_axis_name="s")
    f = pl.kernel(
        _body,
        out_type=jax.ShapeDtypeStruct((B,), jnp.float32),
        mesh=mesh,
        compiler_params=pltpu.CompilerParams(needs_layout_passes=False, use_tc_tiling_on_sc=False),
        scratch_types=[
            pltpu.VMEM((UPW + 16, ), jnp.int32),
            pltpu.VMEM((NSUB, SUB), jnp.int32),
            pltpu.VMEM((T, D), jnp.float32),
            pltpu.VMEM((UPW, D), jnp.float32),
            pltpu.VMEM((T,), jnp.int32),
            pltpu.VMEM((NU, SUB), jnp.int32),
            pltpu.VMEM((NU, SUB), jnp.int32),
            pltpu.VMEM((UPW, D), jnp.float32),
            pltpu.VMEM((UPW, D), jnp.float32),
            pltpu.VMEM((UPW,), jnp.float32),
            pltpu.SemaphoreType.DMA,
            pltpu.SemaphoreType.DMA,
        ],
    )
    return f(users, pos_items, pos_item_offsets, items, user_table, item_table)
